# baseline (speedup 1.0000x reference)
import functools

import jax
import jax.numpy as jnp
from jax.experimental import pallas as pl
from jax.experimental.pallas import tpu as pltpu

_N = 19
_P = 6


def _gcn_body(x_ref, w_ref, mdT_ref, ms_ref, md_ref, k_ref,
              kloop_ref, w1_ref, b1_ref, w2_ref, b2_ref, o_ref, abd_ref):
    n = _N
    g = x_ref.shape[1]
    d_in = x_ref.shape[3]
    sizes = [_P] * (g // _P)
    if g % _P:
        sizes.append(g % _P)

    w = w_ref[0]
    deg = jnp.dot(w, mdT_ref[...],
                  preferred_element_type=jnp.float32) + 1.0
    dis = jnp.where(deg > 0,
                    jax.lax.rsqrt(jnp.maximum(deg, 1e-12)),
                    0.0)
    dis_s = jnp.dot(dis, ms_ref[...],
                    preferred_element_type=jnp.float32)
    dis_d = jnp.dot(dis, md_ref[...],
                    preferred_element_type=jnp.float32)
    norm = dis_s * w * dis_d

    a_flat = (jnp.dot(norm.astype(jnp.bfloat16),
                      k_ref[...].astype(jnp.bfloat16),
                      preferred_element_type=jnp.float32)
              + jnp.dot(dis * dis, kloop_ref[...],
                        preferred_element_type=jnp.float32))

    abd_ref[...] = jnp.zeros(abd_ref.shape, dtype=jnp.float32)
    off = 0
    for t, p in enumerate(sizes):
        a3 = a_flat[off:off + p].reshape(p, n, n)
        for q in range(p):
            abd_ref[t, n * q:n * (q + 1), n * q:n * (q + 1)] = a3[q]
        off += p

    x = x_ref[0].astype(jnp.bfloat16)
    w1 = w1_ref[...].astype(jnp.bfloat16)
    b1 = b1_ref[...]
    w2 = w2_ref[...].astype(jnp.bfloat16)
    b2 = b2_ref[...]
    off = 0
    for t, p in enumerate(sizes):
        rows = p * n
        a_t = abd_ref[t, 0:rows, 0:rows].astype(jnp.bfloat16)
        x_t = x[off:off + p].reshape(rows, d_in)
        z = jnp.dot(a_t, x_t, preferred_element_type=jnp.float32)
        h = jax.nn.sigmoid(jnp.dot(z.astype(jnp.bfloat16), w1,
                                   preferred_element_type=jnp.float32)
                           + b1)
        z2 = jnp.dot(a_t, h.astype(jnp.bfloat16),
                     preferred_element_type=jnp.float32)
        y = jax.nn.sigmoid(jnp.dot(z2.astype(jnp.bfloat16), w2,
                                   preferred_element_type=jnp.float32)
                           + b2)
        o_ref[0, off:off + p, :, :] = y.reshape(p, n, y.shape[1])
        off += p


@functools.partial(jax.jit, static_argnames=("interpret",))
def _run(feature_all, gw, mdT, ms, md, kmat, kloop, W1, b1, W2, b2,
         interpret=False):
    n = _N
    Bb, Tt = feature_all.shape[0], feature_all.shape[1]
    d_in = feature_all.shape[3]
    d_out = W2.shape[1]
    e_cnt = gw.shape[2]
    ngrp = (Tt + _P - 1) // _P

    out = pl.pallas_call(
        _gcn_body,
        grid=(Bb,),
        in_specs=[
            pl.BlockSpec((1, Tt, n, d_in), lambda i: (i, 0, 0, 0)),
            pl.BlockSpec((1, Tt, e_cnt), lambda i: (i, 0, 0)),
            pl.BlockSpec(mdT.shape, lambda i: (0, 0)),
            pl.BlockSpec(ms.shape, lambda i: (0, 0)),
            pl.BlockSpec(md.shape, lambda i: (0, 0)),
            pl.BlockSpec(kmat.shape, lambda i: (0, 0)),
            pl.BlockSpec(kloop.shape, lambda i: (0, 0)),
            pl.BlockSpec(W1.shape, lambda i: (0, 0)),
            pl.BlockSpec(b1.shape, lambda i: (0, 0)),
            pl.BlockSpec(W2.shape, lambda i: (0, 0)),
            pl.BlockSpec(b2.shape, lambda i: (0, 0)),
        ],
        out_specs=pl.BlockSpec((1, Tt, n, d_out), lambda i: (i, 0, 0, 0)),
        out_shape=jax.ShapeDtypeStruct((Bb, Tt, n, d_out), jnp.float32),
        scratch_shapes=[
            pltpu.VMEM((ngrp, _P * n, _P * n), jnp.float32)],
        compiler_params=pltpu.CompilerParams(
            dimension_semantics=("arbitrary",)),
        interpret=interpret,
    )(feature_all, gw, mdT, ms, md, kmat, kloop, W1, b1, W2, b2)
    return out


def kernel(feature_all, graph_index, graph_weight, W1, b1, W2, b2):
    n = feature_all.shape[2]
    src = graph_index[0, 0]
    dst = graph_index[0, 1]
    msT = jax.nn.one_hot(src, n, dtype=jnp.float32)
    mdT = jax.nn.one_hot(dst, n, dtype=jnp.float32)
    kmat = (mdT[:, :, None] * msT[:, None, :]).reshape(src.shape[0], n * n)
    kloop = (jnp.eye(n, dtype=jnp.float32)[:, :, None]
             * jnp.eye(n, dtype=jnp.float32)[:, None, :]).reshape(n, n * n)

    return _run(feature_all, graph_weight, mdT, msT.T, mdT.T, kmat, kloop,
                W1, b1.reshape(1, -1), W2, b2.reshape(1, -1))


# speedup vs baseline: 1.1058x; 1.1058x over previous
import functools

import jax
import jax.numpy as jnp
from jax.experimental import pallas as pl
from jax.experimental.pallas import tpu as pltpu

_N = 19
_P = 6
_GCHUNK = 192


def _gcn_body(g_total, x_ref, w_ref, mdT_ref, ms_ref, md_ref, k_ref,
              kloop_ref, w1_ref, b1_ref, w2_ref, b2_ref, o_ref, abd_ref):
    n = _N
    g = _GCHUNK
    ngrp = g // _P
    rows = _P * n

    valid = g_total - pl.program_id(0) * g
    gmask = (jax.lax.broadcasted_iota(jnp.int32, (g, 1), 0) < valid)
    w = jnp.where(gmask, w_ref[...], 0.0)

    deg = jnp.dot(w, mdT_ref[...],
                  preferred_element_type=jnp.float32) + 1.0
    dis = jnp.where(deg > 0,
                    jax.lax.rsqrt(jnp.maximum(deg, 1e-12)),
                    0.0)
    dis_s = jnp.dot(dis, ms_ref[...],
                    preferred_element_type=jnp.float32)
    dis_d = jnp.dot(dis, md_ref[...],
                    preferred_element_type=jnp.float32)
    norm = dis_s * w * dis_d

    a_flat = (jnp.dot(norm, k_ref[...],
                      preferred_element_type=jnp.float32)
              + jnp.dot(dis * dis, kloop_ref[...],
                        preferred_element_type=jnp.float32))
    a4 = a_flat.reshape(ngrp, _P, n, n)

    abd_ref[...] = jnp.zeros((ngrp, rows, rows), dtype=jnp.float32)
    for p in range(_P):
        abd_ref[:, n * p:n * (p + 1), n * p:n * (p + 1)] = a4[:, p]

    gmask3 = gmask[:, :, None]
    x = jnp.where(gmask3, x_ref[...], 0.0)
    x3 = x.reshape(ngrp, rows, x_ref.shape[2])
    w1 = w1_ref[...]
    b1 = b1_ref[...]
    w2 = w2_ref[...]
    b2 = b2_ref[...]
    for t in range(ngrp):
        a_t = abd_ref[t]
        z = jnp.dot(a_t, x3[t], preferred_element_type=jnp.float32)
        h = jax.nn.sigmoid(jnp.dot(z, w1, preferred_element_type=jnp.float32)
                           + b1)
        z2 = jnp.dot(a_t, h, preferred_element_type=jnp.float32)
        y = jax.nn.sigmoid(jnp.dot(z2, w2, preferred_element_type=jnp.float32)
                           + b2)
        o_ref[_P * t:_P * (t + 1), :, :] = y.reshape(_P, n, y.shape[1])


@functools.partial(jax.jit, static_argnames=("interpret",))
def _run(x3d, ew, mdT, ms, md, kmat, kloop, W1, b1, W2, b2, interpret=False):
    n = _N
    g_total = ew.shape[0]
    d_in = x3d.shape[2]
    d_out = W2.shape[1]
    grid = (g_total + _GCHUNK - 1) // _GCHUNK

    out = pl.pallas_call(
        functools.partial(_gcn_body, g_total),
        grid=(grid,),
        in_specs=[
            pl.BlockSpec((_GCHUNK, n, d_in), lambda i: (i, 0, 0)),
            pl.BlockSpec((_GCHUNK, ew.shape[1]), lambda i: (i, 0)),
            pl.BlockSpec(mdT.shape, lambda i: (0, 0)),
            pl.BlockSpec(ms.shape, lambda i: (0, 0)),
            pl.BlockSpec(md.shape, lambda i: (0, 0)),
            pl.BlockSpec(kmat.shape, lambda i: (0, 0)),
            pl.BlockSpec(kloop.shape, lambda i: (0, 0)),
            pl.BlockSpec(W1.shape, lambda i: (0, 0)),
            pl.BlockSpec(b1.shape, lambda i: (0, 0)),
            pl.BlockSpec(W2.shape, lambda i: (0, 0)),
            pl.BlockSpec(b2.shape, lambda i: (0, 0)),
        ],
        out_specs=pl.BlockSpec((_GCHUNK, n, d_out), lambda i: (i, 0, 0)),
        out_shape=jax.ShapeDtypeStruct((g_total, n, d_out), jnp.float32),
        scratch_shapes=[
            pltpu.VMEM((_GCHUNK // _P, _P * n, _P * n), jnp.float32)],
        compiler_params=pltpu.CompilerParams(
            dimension_semantics=("arbitrary",)),
        interpret=interpret,
    )(x3d, ew, mdT, ms, md, kmat, kloop, W1, b1, W2, b2)
    return out


def kernel(feature_all, graph_index, graph_weight, W1, b1, W2, b2):
    Bb, Tt, n, d_in = feature_all.shape
    g_total = Bb * Tt
    x3d = feature_all.reshape(g_total, n, d_in)
    ew = graph_weight.reshape(g_total, -1)

    src = graph_index[0, 0]
    dst = graph_index[0, 1]
    msT = jax.nn.one_hot(src, n, dtype=jnp.float32)
    mdT = jax.nn.one_hot(dst, n, dtype=jnp.float32)
    kmat = (mdT[:, :, None] * msT[:, None, :]).reshape(src.shape[0], n * n)
    kloop = (jnp.eye(n, dtype=jnp.float32)[:, :, None]
             * jnp.eye(n, dtype=jnp.float32)[:, None, :]).reshape(n, n * n)

    out = _run(x3d, ew, mdT, msT.T, mdT.T, kmat, kloop,
               W1, b1.reshape(1, -1), W2, b2.reshape(1, -1))
    return out.reshape(Bb, Tt, n, W2.shape[1])
